# Initial kernel scaffold; baseline (speedup 1.0000x reference)
import jax
import jax.numpy as jnp
from jax.experimental import pallas as pl


def kernel(key_soft, emb_weight):
    raise NotImplementedError("write your pallas kernel here")


# speedup vs baseline: 1.1366x; 1.1366x over previous
import functools

import jax
import jax.numpy as jnp
from jax.experimental import pallas as pl

N_E = 1024
E_DIM = 64
LEGACY_CLUSTER = 0.2
LEGACY_ENERGY = 0.2
B = 16
T = 576
N_COLS = N_E + 1
N_PAD = 1152
ROW_BLK = 128
N_ROWS = B * T
GRID = N_ROWS // ROW_BLK


def _dist_kernel(ks_ref, emb_ref, d_ref, amin_ref):
    ks = ks_ref[...]
    emb = emb_ref[...]
    rq = jnp.sum(ks * ks, axis=1, keepdims=True)
    ebs = jnp.sum(emb * emb, axis=1)
    col = jax.lax.broadcasted_iota(jnp.int32, (ROW_BLK, N_PAD), 1)
    ebs = jnp.where(col[0:1, :] >= N_COLS, jnp.inf, ebs[None, :])
    m = jax.lax.dot_general(ks, emb, (((1,), (1,)), ((), ())),
                            preferred_element_type=jnp.float32)
    d = (rq + ebs) - 2.0 * m
    d_ref[...] = d
    minv = jnp.min(d, axis=1, keepdims=True)
    idx = jnp.where(d == minv, col, jnp.int32(2 ** 30))
    amin_ref[...] = jnp.min(idx, axis=1).reshape(1, 1, ROW_BLK)


@jax.jit
def _stage_a(ksf, emb_pad):
    d, amin = pl.pallas_call(
        _dist_kernel,
        grid=(GRID,),
        in_specs=[
            pl.BlockSpec((ROW_BLK, E_DIM), lambda i: (i, 0)),
            pl.BlockSpec((N_PAD, E_DIM), lambda i: (0, 0)),
        ],
        out_specs=[
            pl.BlockSpec((ROW_BLK, N_PAD), lambda i: (i, 0)),
            pl.BlockSpec((1, 1, ROW_BLK), lambda i: (i, 0, 0)),
        ],
        out_shape=[
            jax.ShapeDtypeStruct((N_ROWS, N_PAD), jnp.float32),
            jax.ShapeDtypeStruct((GRID, 1, ROW_BLK), jnp.int32),
        ],
    )(ksf, emb_pad)
    return d, amin.reshape(N_ROWS)


def kernel(key_soft, emb_weight):
    sg = jax.lax.stop_gradient
    ksf = key_soft.reshape(-1, E_DIM)
    emb_pad = jnp.pad(emb_weight, ((0, N_PAD - N_COLS), (0, 0)))
    d_pad, min_indices = _stage_a(ksf, emb_pad)
    d_ng = d_pad[:, :N_COLS]

    d3 = d_ng.reshape(B, T, N_E + 1)
    ind0 = jnp.clip(min_indices.reshape(B, T)[:, 0], 0, N_E - 1)

    def step(ind, d_t):
        d_here = jnp.take_along_axis(d_t, ind[:, None], axis=1)[:, 0]
        ind_next = jnp.clip(ind + 1, 0, N_E - 1)
        d_next = jnp.take_along_axis(d_t, ind_next[:, None], axis=1)[:, 0]
        ind_new = jnp.where(d_here <= d_next, ind, ind_next)
        return ind_new, ind_new

    _, inds_rest = jax.lax.scan(step, ind0, jnp.swapaxes(d3[:, 1:, :], 0, 1))
    encoding_indices = jnp.concatenate(
        [ind0[:, None], jnp.swapaxes(inds_rest, 0, 1)], axis=1)
    flat = encoding_indices.reshape(-1)
    key_hard_here = emb_weight[flat].reshape(key_soft.shape)
    key_hard_next = emb_weight[flat + 1].reshape(key_soft.shape)
    key_min = emb_weight[min_indices].reshape(key_soft.shape)
    key_em_here = jnp.sum((sg(key_soft) - key_hard_here) ** 2, axis=-1) + \
        jnp.sum((key_soft - sg(key_hard_here)) ** 2, axis=-1) * LEGACY_ENERGY
    key_em_next = jnp.sum((sg(key_soft) - key_hard_next) ** 2, axis=-1) + \
        jnp.sum((key_soft - sg(key_hard_next)) ** 2, axis=-1) * LEGACY_ENERGY
    key_energy_mat = key_em_next - key_em_here
    indices_change = (encoding_indices[:, 1:] - encoding_indices[:, :-1]).astype(bool)
    key_energy_change = key_energy_mat[:, 1:] - key_energy_mat[:, :-1]
    same_hard_mask = jnp.where(indices_change, 0.0, 1.0)
    key_energy_change = key_energy_change * same_hard_mask
    loss_key_energy_descent = jnp.maximum(
        key_energy_change + 1e-06 / N_E, jnp.zeros_like(key_energy_change)).mean()
    loss_min_indices = jnp.sum((sg(key_soft) - key_min) ** 2, axis=-1) + \
        jnp.sum((key_soft - sg(key_min)) ** 2, axis=-1) * LEGACY_CLUSTER
    reg_persist_mat = jnp.exp(-key_em_next)
    e_normal_mat = jnp.where(
        key_em_here > loss_min_indices - 1e-06 / N_E,
        key_em_here - loss_min_indices + 1e-06 / N_E, key_em_here) + reg_persist_mat
    reg_escape_mat = jnp.exp(-key_em_here)
    e_abnormal_mat = key_em_next + reg_escape_mat
    key_hard = key_soft + sg(key_hard_here - key_soft)
    mn = jnp.min(encoding_indices, axis=1)
    mx = jnp.max(encoding_indices, axis=1)
    v = jnp.max(mx - mn)
    return (key_hard, encoding_indices, v, loss_key_energy_descent,
            key_energy_mat, e_normal_mat, e_abnormal_mat)


# speedup vs baseline: 4.2094x; 3.7034x over previous
import functools

import jax
import jax.numpy as jnp
from jax import lax
from jax.experimental import pallas as pl
from jax.experimental.pallas import tpu as pltpu
from jax.experimental.pallas import tpu_sc as plsc

N_E = 1024
E_DIM = 64
LEGACY_CLUSTER = 0.2
LEGACY_ENERGY = 0.2
B = 16
T = 576
N_COLS = N_E + 1
N_PAD = 1152
ROW_BLK = 128
N_ROWS = B * T
GRID = N_ROWS // ROW_BLK

WIN_W = 768
WIN_C0_MAX = N_PAD - WIN_W
SEG = 144
SEG_STAGE = 152

N_WORKERS = 32
ROWS_PER_W = N_ROWS // N_WORKERS


def _dist_kernel(ks_ref, emb_ref, d_ref, amin_ref):
    ks = ks_ref[...]
    emb = emb_ref[...]
    rq = jnp.sum(ks * ks, axis=1, keepdims=True)
    ebs = jnp.sum(emb * emb, axis=1)
    col = jax.lax.broadcasted_iota(jnp.int32, (ROW_BLK, N_PAD), 1)
    ebs = jnp.where(col[0:1, :] >= N_COLS, jnp.inf, ebs[None, :])
    m = jax.lax.dot_general(ks, emb, (((1,), (1,)), ((), ())),
                            preferred_element_type=jnp.float32)
    d = (rq + ebs) - 2.0 * m
    d_ref[...] = d
    minv = jnp.min(d, axis=1, keepdims=True)
    idx = jnp.where(d == minv, col, jnp.int32(2 ** 30))
    amin_ref[...] = jnp.min(idx, axis=1).reshape(1, 1, ROW_BLK)


def _scan_body(d_hbm, ind0_hbm, enc_hbm, win, encbuf, ind0buf):
    c = lax.axis_index("c")
    s = lax.axis_index("s")
    lane = jax.lax.broadcasted_iota(jnp.int32, (16,), 0)

    @pl.when(c == 0)
    def _():
        b = s
        pltpu.sync_copy(ind0_hbm.at[pl.ds(b * 16, 16)], ind0buf)
        ind0 = ind0buf[...][0]
        c0 = jnp.minimum((ind0 // 128) * 128, WIN_C0_MAX)
        p0 = ind0
        acc0 = jnp.where(lane == 0, jnp.full((16,), p0, jnp.int32),
                         jnp.zeros((16,), jnp.int32))
        neginf = jnp.full((16,), -jnp.inf, jnp.float32)

        def run_seg(carry, stage0, nstage, t0, nsteps):
            pltpu.sync_copy(
                d_hbm.at[pl.ds(b * T + stage0, nstage), pl.ds(c0, WIN_W)],
                win.at[pl.ds(0, nstage), :])
            off = t0 - stage0

            def step(i, carry):
                p, acc = carry
                e = t0 + i
                row = off + i
                pc = p - c0
                base = (pc // 16) * 16
                r = pc - base
                v0 = win[row, pl.ds(base, 16)]
                v1 = win[row, pl.ds(base + 16, 16)]
                dh = jnp.max(jnp.where(lane == r, v0, neginf))
                dnA = jnp.max(jnp.where(lane == r + 1, v0, neginf))
                dnB = jnp.max(jnp.where(lane == 0, v1, neginf))
                dn = jnp.where(r == 15, dnB, dnA)
                pn = jnp.minimum(p + 1, N_E - 1)
                dn = jnp.where(p == N_E - 1, dh, dn)
                p_new = jnp.where(dn < dh, pn, p)
                acc = jnp.where(lane == e % 16,
                                jnp.full((16,), p_new, jnp.int32), acc)

                @pl.when(e % 16 == 15)
                def _():
                    encbuf[pl.ds(e - 15, 16)] = acc

                return (p_new, acc)

            return lax.fori_loop(0, nsteps, step, carry)

        carry = (p0, acc0)
        carry = run_seg(carry, 0, SEG_STAGE, 1, SEG)
        carry = run_seg(carry, SEG, SEG_STAGE, 1 + SEG, SEG)
        carry = run_seg(carry, 2 * SEG, SEG_STAGE, 1 + 2 * SEG, SEG)
        carry = run_seg(carry, 3 * SEG, SEG, 1 + 3 * SEG, T - 1 - 3 * SEG)
        pltpu.sync_copy(encbuf, enc_hbm.at[pl.ds(b * T, T)])


def _gather_body(emb_hbm, enc_hbm, encp1_hbm, amin_hbm, kh_hbm, khn_hbm,
                 km_hbm, idxbuf, rows, sem):
    c = lax.axis_index("c")
    s = lax.axis_index("s")
    w = s * 2 + c
    base = w * ROWS_PER_W
    for src, dst in ((enc_hbm, kh_hbm), (encp1_hbm, khn_hbm),
                     (amin_hbm, km_hbm)):
        pltpu.sync_copy(src.at[pl.ds(base, ROWS_PER_W)], idxbuf)
        pltpu.async_copy(emb_hbm.at[idxbuf], rows, sem).wait()
        pltpu.sync_copy(rows, dst.at[pl.ds(base, ROWS_PER_W)])


@jax.jit
def _pallas_pipeline(ksf, emb_pad, emb128):
    d, amin = pl.pallas_call(
        _dist_kernel,
        grid=(GRID,),
        in_specs=[
            pl.BlockSpec((ROW_BLK, E_DIM), lambda i: (i, 0)),
            pl.BlockSpec((N_PAD, E_DIM), lambda i: (0, 0)),
        ],
        out_specs=[
            pl.BlockSpec((ROW_BLK, N_PAD), lambda i: (i, 0)),
            pl.BlockSpec((1, 1, ROW_BLK), lambda i: (i, 0, 0)),
        ],
        out_shape=[
            jax.ShapeDtypeStruct((N_ROWS, N_PAD), jnp.float32),
            jax.ShapeDtypeStruct((GRID, 1, ROW_BLK), jnp.int32),
        ],
    )(ksf, emb_pad)
    amin = amin.reshape(N_ROWS)

    ind0 = jnp.clip(amin.reshape(B, T)[:, 0], 0, N_E - 1)
    ind0_rep = jnp.repeat(ind0, 16)

    mesh = plsc.VectorSubcoreMesh(core_axis_name="c", subcore_axis_name="s")
    sc_params = pltpu.CompilerParams(needs_layout_passes=False)
    scan_call = pl.kernel(
        _scan_body, mesh=mesh, compiler_params=sc_params,
        out_type=jax.ShapeDtypeStruct((N_ROWS,), jnp.int32),
        scratch_types=[
            pltpu.VMEM((SEG_STAGE, WIN_W), jnp.float32),
            pltpu.VMEM((T,), jnp.int32),
            pltpu.VMEM((16,), jnp.int32),
        ],
    )
    enc_flat = scan_call(d, ind0_rep)

    encp1 = enc_flat + 1
    gather_call = pl.kernel(
        _gather_body, mesh=mesh, compiler_params=sc_params,
        out_type=[jax.ShapeDtypeStruct((N_ROWS, 128), jnp.float32)] * 3,
        scratch_types=[
            pltpu.VMEM((ROWS_PER_W,), jnp.int32),
            pltpu.VMEM((ROWS_PER_W, 128), jnp.float32),
            pltpu.SemaphoreType.DMA,
        ],
    )
    kh, khn, km = gather_call(emb128, enc_flat, encp1, amin)
    return enc_flat, kh, khn, km


def kernel(key_soft, emb_weight):
    sg = jax.lax.stop_gradient
    ksf = key_soft.reshape(-1, E_DIM)
    emb_pad = jnp.pad(emb_weight, ((0, N_PAD - N_COLS), (0, 0)))
    emb128 = jnp.pad(emb_weight, ((0, 0), (0, 128 - E_DIM)))
    enc_flat, kh, khn, km = _pallas_pipeline(ksf, emb_pad, emb128)
    kh, khn, km = kh[:, :E_DIM], khn[:, :E_DIM], km[:, :E_DIM]

    encoding_indices = enc_flat.reshape(B, T)
    key_hard_here = kh.reshape(key_soft.shape)
    key_hard_next = khn.reshape(key_soft.shape)
    key_min = km.reshape(key_soft.shape)
    key_em_here = jnp.sum((sg(key_soft) - key_hard_here) ** 2, axis=-1) + \
        jnp.sum((key_soft - sg(key_hard_here)) ** 2, axis=-1) * LEGACY_ENERGY
    key_em_next = jnp.sum((sg(key_soft) - key_hard_next) ** 2, axis=-1) + \
        jnp.sum((key_soft - sg(key_hard_next)) ** 2, axis=-1) * LEGACY_ENERGY
    key_energy_mat = key_em_next - key_em_here
    indices_change = (encoding_indices[:, 1:] - encoding_indices[:, :-1]).astype(bool)
    key_energy_change = key_energy_mat[:, 1:] - key_energy_mat[:, :-1]
    same_hard_mask = jnp.where(indices_change, 0.0, 1.0)
    key_energy_change = key_energy_change * same_hard_mask
    loss_key_energy_descent = jnp.maximum(
        key_energy_change + 1e-06 / N_E, jnp.zeros_like(key_energy_change)).mean()
    loss_min_indices = jnp.sum((sg(key_soft) - key_min) ** 2, axis=-1) + \
        jnp.sum((key_soft - sg(key_min)) ** 2, axis=-1) * LEGACY_CLUSTER
    reg_persist_mat = jnp.exp(-key_em_next)
    e_normal_mat = jnp.where(
        key_em_here > loss_min_indices - 1e-06 / N_E,
        key_em_here - loss_min_indices + 1e-06 / N_E, key_em_here) + reg_persist_mat
    reg_escape_mat = jnp.exp(-key_em_here)
    e_abnormal_mat = key_em_next + reg_escape_mat
    key_hard = key_soft + sg(key_hard_here - key_soft)
    mn = jnp.min(encoding_indices, axis=1)
    mx = jnp.max(encoding_indices, axis=1)
    v = jnp.max(mx - mn)
    return (key_hard, encoding_indices, v, loss_key_energy_descent,
            key_energy_mat, e_normal_mat, e_abnormal_mat)
